# baseline (speedup 1.0000x reference)
import functools

import jax
import jax.numpy as jnp
from jax import lax
from jax.experimental import pallas as pl
from jax.experimental.pallas import tpu as pltpu
from jax.experimental.pallas import tpu_sc as plsc

E = 16
CAP = 320
BS = 1024
NS = 16
TOK = 4096 // NS


def _tc_body(x_ref, w_ref, rp_ref, lg_ref):
    x = x_ref[0]
    w = w_ref[...]
    logits = jax.lax.dot_general(
        x, w, (((1,), (1,)), ((), ())), preferred_element_type=jnp.float32
    )
    lg_ref[0] = logits
    m = jnp.max(logits, axis=-1, keepdims=True)
    denom = jnp.sum(jnp.exp(logits - m), axis=-1, keepdims=True)
    rp_ref[0] = 1.0 / denom


def _sc_body(lg_hbm, ei_hbm, tot_hbm, lg_v, pri_v, oh_v, ei_v, tot_v, all_v):
    c = lax.axis_index("c")
    s = lax.axis_index("s")

    pltpu.sync_copy(lg_hbm.at[c, pl.ds(s * TOK * E, TOK * E)], lg_v)

    lanes = lax.iota(jnp.int32, E)

    def scan_tok(t, counts):
        lg = lg_v[pl.ds(t * E, E)]
        m = jnp.max(lg)
        first = plsc.all_reduce_ffs(lg == m)
        oh = jnp.where(lanes == first, 1, 0)
        cnt = counts + oh
        pri_v[pl.ds(t * E, E)] = cnt
        oh_v[pl.ds(t * E, E)] = oh
        return cnt

    totals = lax.fori_loop(0, TOK, scan_tok, jnp.zeros((E,), jnp.int32))

    tot_v[...] = totals
    pltpu.sync_copy(tot_v, tot_hbm.at[c, pl.ds(s * E, E)])
    plsc.subcore_barrier()
    pltpu.sync_copy(tot_hbm.at[c], all_v)

    def add_prev(i, off):
        return off + jnp.where(i < s, all_v[pl.ds(i * E, E)], 0)

    offset = lax.fori_loop(0, NS, add_prev, jnp.zeros((E,), jnp.int32))

    def mask_tok(t, _):
        pri = pri_v[pl.ds(t * E, E)] + offset
        keep = jnp.where(pri <= CAP, oh_v[pl.ds(t * E, E)], 0)
        ei_v[pl.ds(t * E, E)] = keep
        return 0

    lax.fori_loop(0, TOK, mask_tok, 0)
    pltpu.sync_copy(ei_v, ei_hbm.at[c, pl.ds(s * TOK * E, TOK * E)])


def _sc_route(lg):
    B, S, _ = lg.shape
    lg_flat = lg.reshape(B, S * E)
    mesh = plsc.VectorSubcoreMesh(core_axis_name="c", subcore_axis_name="s")
    fn = functools.partial(
        pl.kernel,
        out_type=(
            jax.ShapeDtypeStruct((B, S * E), jnp.int32),
            jax.ShapeDtypeStruct((B, NS * E), jnp.int32),
        ),
        mesh=mesh,
        scratch_types=[
            pltpu.VMEM((TOK * E,), jnp.float32),
            pltpu.VMEM((TOK * E,), jnp.int32),
            pltpu.VMEM((TOK * E,), jnp.int32),
            pltpu.VMEM((TOK * E,), jnp.int32),
            pltpu.VMEM((E,), jnp.int32),
            pltpu.VMEM((NS * E,), jnp.int32),
        ],
        compiler_params=pltpu.CompilerParams(needs_layout_passes=False),
    )(_sc_body)
    ei, _ = fn(lg_flat)
    return ei.reshape(B, S, E)


def kernel(x, W):
    B, S, D = x.shape
    grid = (B, S // BS)
    out_shapes = (
        jax.ShapeDtypeStruct((B, S, 1), jnp.float32),
        jax.ShapeDtypeStruct((B, S, E), jnp.float32),
    )
    rp, lg = pl.pallas_call(
        _tc_body,
        grid=grid,
        in_specs=[
            pl.BlockSpec((1, BS, D), lambda b, s: (b, s, 0)),
            pl.BlockSpec((E, D), lambda b, s: (0, 0)),
        ],
        out_specs=(
            pl.BlockSpec((1, BS, 1), lambda b, s: (b, s, 0)),
            pl.BlockSpec((1, BS, E), lambda b, s: (b, s, 0)),
        ),
        out_shape=out_shapes,
        compiler_params=pltpu.CompilerParams(
            dimension_semantics=("arbitrary", "arbitrary"),
        ),
    )(x, W)
    ei = _sc_route(lg)
    return (ei, rp, lg)


# speedup vs baseline: 1.5493x; 1.5493x over previous
import jax
import jax.numpy as jnp
from jax.experimental import pallas as pl
from jax.experimental.pallas import tpu as pltpu

E = 16
CAP = 320
BS = 1024
C = 128


def _router_body(x_ref, w_ref, ei_ref, rp_ref, lg_ref, carry_ref, tri_ref):
    b = pl.program_id(0)
    sb = pl.program_id(1)

    @pl.when((b == 0) & (sb == 0))
    def _():
        row = jax.lax.broadcasted_iota(jnp.int32, (C, C), 0)
        col = jax.lax.broadcasted_iota(jnp.int32, (C, C), 1)
        tri_ref[...] = (row >= col).astype(jnp.float32)

    @pl.when(sb == 0)
    def _():
        carry_ref[...] = jnp.zeros_like(carry_ref)

    x = x_ref[0]
    w = w_ref[...]
    logits = jax.lax.dot_general(
        x, w, (((1,), (1,)), ((), ())), preferred_element_type=jnp.float32
    )
    lg_ref[0] = logits

    m = jnp.max(logits, axis=-1, keepdims=True)
    denom = jnp.sum(jnp.exp(logits - m), axis=-1, keepdims=True)
    rp_ref[0] = 1.0 / denom

    is_max = (logits == m).astype(jnp.float32)
    er = jax.lax.broadcasted_iota(jnp.int32, (E, E), 0)
    ec = jax.lax.broadcasted_iota(jnp.int32, (E, E), 1)
    upper = (er < ec).astype(jnp.float32)
    prior = jax.lax.dot_general(
        is_max, upper, (((1,), (0,)), ((), ())),
        preferred_element_type=jnp.float32,
    )
    one_hot = is_max * (prior == 0.0)

    tri = tri_ref[...]
    offset = carry_ref[...]
    for c in range(BS // C):
        oh_c = one_hot[c * C:(c + 1) * C]
        cs_c = jax.lax.dot_general(
            tri, oh_c, (((1,), (0,)), ((), ())),
            preferred_element_type=jnp.float32,
        )
        priority = cs_c + offset
        keep = priority <= float(CAP)
        ei_ref[0, c * C:(c + 1) * C] = jnp.where(
            keep, oh_c, 0.0
        ).astype(jnp.int32)
        offset = offset + cs_c[C - 1:C]
    carry_ref[...] = offset


def kernel(x, W):
    B, S, D = x.shape
    grid = (B, S // BS)
    out_shapes = (
        jax.ShapeDtypeStruct((B, S, E), jnp.int32),
        jax.ShapeDtypeStruct((B, S, 1), jnp.float32),
        jax.ShapeDtypeStruct((B, S, E), jnp.float32),
    )
    ei, rp, lg = pl.pallas_call(
        _router_body,
        grid=grid,
        in_specs=[
            pl.BlockSpec((1, BS, D), lambda b, s: (b, s, 0)),
            pl.BlockSpec((E, D), lambda b, s: (0, 0)),
        ],
        out_specs=(
            pl.BlockSpec((1, BS, E), lambda b, s: (b, s, 0)),
            pl.BlockSpec((1, BS, 1), lambda b, s: (b, s, 0)),
            pl.BlockSpec((1, BS, E), lambda b, s: (b, s, 0)),
        ),
        out_shape=out_shapes,
        scratch_shapes=[
            pltpu.VMEM((1, E), jnp.float32),
            pltpu.VMEM((C, C), jnp.float32),
        ],
        compiler_params=pltpu.CompilerParams(
            dimension_semantics=("arbitrary", "arbitrary"),
        ),
    )(x, W)
    return (ei, rp, lg)
